# baseline (speedup 1.0000x reference)
import jax
import jax.numpy as jnp
from jax.experimental import pallas as pl
from jax.experimental.pallas import tpu as pltpu


def _mm(a, b, precision):
    return jax.lax.dot_general(
        a, b, (((1,), (0,)), ((), ())),
        precision=precision, preferred_element_type=jnp.float32)


def _xw_body(x_ref, w_ref, o_ref):
    o_ref[...] = _mm(x_ref[...], w_ref[...], jax.lax.Precision.HIGHEST)


def _layer1_body(adj_ref, s1_ref, b1_ref, w2_ref, o_ref):
    t = _mm(adj_ref[...], s1_ref[...], jax.lax.Precision.DEFAULT)
    h = jnp.maximum(t + b1_ref[...], 0.0)
    o_ref[...] = _mm(h, w2_ref[...], jax.lax.Precision.DEFAULT)


def _layer2_body(adj_ref, s2_ref, b2_ref, o_ref):
    t = _mm(adj_ref[...], s2_ref[...], jax.lax.Precision.DEFAULT)
    logits = t + b2_ref[...]
    m = jnp.max(logits, axis=1, keepdims=True)
    e = jnp.exp(logits - m)
    lse = m + jnp.log(jnp.sum(e, axis=1, keepdims=True))
    o_ref[...] = logits - lse


def kernel(raw_x, adj, W1, b1, W2, b2):
    n, nfeat = raw_x.shape
    nhid = W1.shape[1]
    ncls = W2.shape[1]
    b1r = b1.reshape(1, nhid)
    b2r = b2.reshape(1, ncls)

    BX = min(1000, n)
    s1 = pl.pallas_call(
        _xw_body,
        grid=(n // BX,),
        in_specs=[
            pl.BlockSpec((BX, nfeat), lambda i: (i, 0)),
            pl.BlockSpec((nfeat, nhid), lambda i: (0, 0)),
        ],
        out_specs=pl.BlockSpec((BX, nhid), lambda i: (i, 0)),
        out_shape=jax.ShapeDtypeStruct((n, nhid), jnp.float32),
        compiler_params=pltpu.CompilerParams(
            dimension_semantics=("parallel",)),
    )(raw_x, W1)

    BI = min(400, n)
    s2 = pl.pallas_call(
        _layer1_body,
        grid=(n // BI,),
        in_specs=[
            pl.BlockSpec((BI, n), lambda i: (i, 0)),
            pl.BlockSpec((n, nhid), lambda i: (0, 0)),
            pl.BlockSpec((1, nhid), lambda i: (0, 0)),
            pl.BlockSpec((nhid, ncls), lambda i: (0, 0)),
        ],
        out_specs=pl.BlockSpec((BI, ncls), lambda i: (i, 0)),
        out_shape=jax.ShapeDtypeStruct((n, ncls), jnp.float32),
        compiler_params=pltpu.CompilerParams(
            dimension_semantics=("parallel",)),
    )(adj, s1, b1r, W2)

    pred = pl.pallas_call(
        _layer2_body,
        grid=(n // BI,),
        in_specs=[
            pl.BlockSpec((BI, n), lambda i: (i, 0)),
            pl.BlockSpec((n, ncls), lambda i: (0, 0)),
            pl.BlockSpec((1, ncls), lambda i: (0, 0)),
        ],
        out_specs=pl.BlockSpec((BI, ncls), lambda i: (i, 0)),
        out_shape=jax.ShapeDtypeStruct((n, ncls), jnp.float32),
        compiler_params=pltpu.CompilerParams(
            dimension_semantics=("parallel",)),
    )(adj, s2, b2r)
    return pred


# speedup vs baseline: 1.0564x; 1.0564x over previous
import jax
import jax.numpy as jnp
from jax.experimental import pallas as pl
from jax.experimental.pallas import tpu as pltpu


def _mm(a, b, precision):
    return jax.lax.dot_general(
        a, b, (((1,), (0,)), ((), ())),
        precision=precision, preferred_element_type=jnp.float32)


def _layer1_body(x_ref, w1_ref, adj_ref, b1_ref, w2_ref, o_ref, s1_ref):
    @pl.when(pl.program_id(0) == 0)
    def _():
        s1_ref[...] = _mm(x_ref[...], w1_ref[...], jax.lax.Precision.DEFAULT)

    t = _mm(adj_ref[...], s1_ref[...], jax.lax.Precision.DEFAULT)
    h = jnp.maximum(t + b1_ref[...], 0.0)
    o_ref[...] = _mm(h, w2_ref[...], jax.lax.Precision.DEFAULT)


def _layer2_body(adj_ref, s2_ref, b2_ref, o_ref):
    t = _mm(adj_ref[...], s2_ref[...], jax.lax.Precision.DEFAULT)
    logits = t + b2_ref[...]
    m = jnp.max(logits, axis=1, keepdims=True)
    e = jnp.exp(logits - m)
    lse = m + jnp.log(jnp.sum(e, axis=1, keepdims=True))
    o_ref[...] = logits - lse


def kernel(raw_x, adj, W1, b1, W2, b2):
    n, nfeat = raw_x.shape
    nhid = W1.shape[1]
    ncls = W2.shape[1]
    b1r = b1.reshape(1, nhid)
    b2r = b2.reshape(1, ncls)

    BI = min(400, n)

    s2 = pl.pallas_call(
        _layer1_body,
        grid=(n // BI,),
        in_specs=[
            pl.BlockSpec((n, nfeat), lambda i: (0, 0)),
            pl.BlockSpec((nfeat, nhid), lambda i: (0, 0)),
            pl.BlockSpec((BI, n), lambda i: (i, 0)),
            pl.BlockSpec((1, nhid), lambda i: (0, 0)),
            pl.BlockSpec((nhid, ncls), lambda i: (0, 0)),
        ],
        out_specs=pl.BlockSpec((BI, ncls), lambda i: (i, 0)),
        out_shape=jax.ShapeDtypeStruct((n, ncls), jnp.float32),
        scratch_shapes=[pltpu.VMEM((n, nhid), jnp.float32)],
        compiler_params=pltpu.CompilerParams(
            dimension_semantics=("arbitrary",)),
    )(raw_x, W1, adj, b1r, W2)

    pred = pl.pallas_call(
        _layer2_body,
        grid=(n // BI,),
        in_specs=[
            pl.BlockSpec((BI, n), lambda i: (i, 0)),
            pl.BlockSpec((n, ncls), lambda i: (0, 0)),
            pl.BlockSpec((1, ncls), lambda i: (0, 0)),
        ],
        out_specs=pl.BlockSpec((BI, ncls), lambda i: (i, 0)),
        out_shape=jax.ShapeDtypeStruct((n, ncls), jnp.float32),
        compiler_params=pltpu.CompilerParams(
            dimension_semantics=("parallel",)),
    )(adj, s2, b2r)
    return pred


# speedup vs baseline: 1.0852x; 1.0273x over previous
import jax
import jax.numpy as jnp
from jax.experimental import pallas as pl
from jax.experimental.pallas import tpu as pltpu


def _mm(a, b):
    return jax.lax.dot_general(
        a, b, (((1,), (0,)), ((), ())),
        precision=jax.lax.Precision.DEFAULT,
        preferred_element_type=jnp.float32)


def _body(x_ref, w1_ref, adj_ref, b1_ref, w2_ref, b2_ref, o_ref,
          s1_ref, s2_ref):
    i = pl.program_id(0)
    nsteps = pl.num_programs(0)
    p = nsteps // 2
    bi = adj_ref.shape[0]

    @pl.when(i == 0)
    def _():
        s1_ref[...] = _mm(x_ref[...], w1_ref[...])

    @pl.when(i < p)
    def _():
        t = _mm(adj_ref[...], s1_ref[...])
        h = jnp.maximum(t + b1_ref[...], 0.0)
        s2_ref[pl.ds(i * bi, bi), :] = _mm(h, w2_ref[...])

    @pl.when(i >= p)
    def _():
        t = _mm(adj_ref[...], s2_ref[...])
        logits = t + b2_ref[...]
        m = jnp.max(logits, axis=1, keepdims=True)
        e = jnp.exp(logits - m)
        lse = m + jnp.log(jnp.sum(e, axis=1, keepdims=True))
        o_ref[...] = logits - lse


def kernel(raw_x, adj, W1, b1, W2, b2):
    n, nfeat = raw_x.shape
    nhid = W1.shape[1]
    ncls = W2.shape[1]
    b1r = b1.reshape(1, nhid)
    b2r = b2.reshape(1, ncls)

    BI = min(400, n)
    P = n // BI

    adj_map = lambda i: (jax.lax.rem(i, P), 0)
    const = lambda i: (0, 0)

    pred = pl.pallas_call(
        _body,
        grid=(2 * P,),
        in_specs=[
            pl.BlockSpec((n, nfeat), const),
            pl.BlockSpec((nfeat, nhid), const),
            pl.BlockSpec((BI, n), adj_map),
            pl.BlockSpec((1, nhid), const),
            pl.BlockSpec((nhid, ncls), const),
            pl.BlockSpec((1, ncls), const),
        ],
        out_specs=pl.BlockSpec(
            (BI, ncls), lambda i: (jnp.where(i < P, 0, i - P), 0)),
        out_shape=jax.ShapeDtypeStruct((n, ncls), jnp.float32),
        scratch_shapes=[
            pltpu.VMEM((n, nhid), jnp.float32),
            pltpu.VMEM((n, ncls), jnp.float32),
        ],
        compiler_params=pltpu.CompilerParams(
            dimension_semantics=("arbitrary",)),
    )(raw_x, W1, adj, b1r, W2, b2r)
    return pred
